# baseline (speedup 1.0000x reference)
import functools

import jax
import jax.numpy as jnp
from jax import lax
from jax.experimental import pallas as pl
from jax.experimental.pallas import tpu as pltpu
from jax.experimental.pallas import tpu_sc as plsc

NC = 2
NS = 16
L = 16
NW = NC * NS

IDX_CHUNK = 128


def _make_bpr(B, V_user, V_item, D):
    bpw = B // NW
    n_chunks = bpw // IDX_CHUNK
    mesh = plsc.VectorSubcoreMesh(core_axis_name="c", subcore_axis_name="s")

    @functools.partial(
        pl.kernel,
        out_type=jax.ShapeDtypeStruct((NW, L), jnp.float32),
        mesh=mesh,
        compiler_params=pltpu.CompilerParams(needs_layout_passes=False,
                                             use_tc_tiling_on_sc=False),
        scratch_types=[
            pltpu.VMEM((n_chunks, IDX_CHUNK), jnp.int32),
            pltpu.VMEM((n_chunks, IDX_CHUNK), jnp.int32),
            pltpu.VMEM((bpw,), jnp.float32),
            pltpu.VMEM((bpw, D), jnp.float32),
            pltpu.VMEM((bpw, D), jnp.float32),
            pltpu.VMEM((L,), jnp.float32),
            pltpu.SemaphoreType.DMA,
        ],
    )
    def bpr(users_hbm, items_hbm, scores_hbm, ut_hbm, it_hbm, out_hbm,
            uidx, iidx, sc_v, urows, irows, acc_v, sem):
        cid = lax.axis_index("c")
        sid = lax.axis_index("s")
        wid = sid * NC + cid
        base = wid * bpw

        for j in range(n_chunks):
            pltpu.sync_copy(users_hbm.at[pl.ds(base + j * IDX_CHUNK, IDX_CHUNK)],
                            uidx.at[j])
            pltpu.sync_copy(items_hbm.at[pl.ds(base + j * IDX_CHUNK, IDX_CHUNK)],
                            iidx.at[j])
        pltpu.sync_copy(scores_hbm.at[pl.ds(base, bpw)], sc_v)

        copies = []
        for j in range(n_chunks):
            copies.append(pltpu.async_copy(
                ut_hbm.at[uidx.at[j]],
                urows.at[pl.ds(j * IDX_CHUNK, IDX_CHUNK)], sem))
            copies.append(pltpu.async_copy(
                it_hbm.at[iidx.at[j]],
                irows.at[pl.ds(j * IDX_CHUNK, IDX_CHUNK)], sem))
        for cp in copies:
            cp.wait()

        def chunk_body(c, acc):
            r0 = c * L
            scv = sc_v[pl.ds(r0, L)]
            for k in range(L):
                r = r0 + k
                prod = jnp.zeros((L,), jnp.float32)
                for d0 in range(0, D, L):
                    prod = prod + (urows[r, pl.ds(d0, L)] *
                                   irows[r, pl.ds(d0, L)])
                diff = jnp.sum(prod) - scv[k]
                acc = acc + diff * diff
            return acc

        acc = lax.fori_loop(0, bpw // L, chunk_body, jnp.float32(0.0))
        lane = lax.iota(jnp.int32, L)
        acc_v[...] = jnp.where(lane == 0, acc, jnp.float32(0.0))
        pltpu.sync_copy(acc_v, out_hbm.at[wid])

    return bpr


def kernel(users, items, scores, user_table, item_table):
    B = users.shape[0]
    V_user, D = user_table.shape
    V_item = item_table.shape[0]
    bpr = _make_bpr(B, V_user, V_item, D)
    partials = bpr(users.astype(jnp.int32), items.astype(jnp.int32),
                   scores, user_table, item_table)
    return jnp.sum(partials) / B
